# Initial kernel scaffold; baseline (speedup 1.0000x reference)
import jax
import jax.numpy as jnp
from jax.experimental import pallas as pl


def kernel(obs, actions, rewards, bootstrap, discount, q_support, W1, b1, g1, be1, W2, b2, g2, be2, W3, b3, g3, be3, W4, b4):
    raise NotImplementedError("write your pallas kernel here")


# speedup vs baseline: 53.8727x; 53.8727x over previous
import functools

import jax
import jax.numpy as jnp
from jax.experimental import pallas as pl
from jax.experimental.pallas import tpu as pltpu

_V_MIN = -10.0
_V_MAX = 10.0
_ATOMS = 251
_AP = 256
_ROWS = 256


def _take_lane(tbl, idx):
    idx_lo = jnp.bitwise_and(idx, 127)
    t0 = jnp.take_along_axis(tbl[:, :128], idx_lo, axis=1)
    t1 = jnp.take_along_axis(tbl[:, 128:], idx_lo, axis=1)
    return jnp.where(idx < 128, t0, t1)


def _cumsum_lane(x, lane_iota):
    for sh in (1, 2, 4, 8, 16, 32, 64, 128):
        rolled = pltpu.roll(x, sh, axis=1)
        x = x + jnp.where(lane_iota >= sh, rolled, 0.0)
    return x


def _ln_silu(h, vecs):
    h = h + vecs[0:1, :]
    m = jnp.mean(h, axis=-1, keepdims=True)
    d = h - m
    v = jnp.mean(d * d, axis=-1, keepdims=True)
    h = d * jax.lax.rsqrt(v + 1e-5) * vecs[1:2, :] + vecs[2:3, :]
    return h * (1.0 / (1.0 + jnp.exp(-h)))


def _block_kernel(obs_ref, act_ref, aux_ref, qsup_ref,
                  w1_ref, v1_ref, w2_ref, v2_ref, w3_ref, v3_ref,
                  w4_ref, b4_ref, out_ref):
    f32 = jnp.float32
    delta_z = (_V_MAX - _V_MIN) / (_ATOMS - 1)

    x = jnp.dot(obs_ref[...], w1_ref[:128, :], preferred_element_type=f32)
    x = x + jnp.dot(act_ref[...], w1_ref[128:, :], preferred_element_type=f32)
    x = _ln_silu(x, v1_ref[...])
    x = _ln_silu(jnp.dot(x, w2_ref[...], preferred_element_type=f32), v2_ref[...])
    x = _ln_silu(jnp.dot(x, w3_ref[...], preferred_element_type=f32), v3_ref[...])
    logits = jnp.dot(x, w4_ref[...], preferred_element_type=f32) + b4_ref[...]

    mx = jnp.max(logits, axis=-1, keepdims=True)
    e = jnp.exp(logits - mx)
    p = e / jnp.sum(e, axis=-1, keepdims=True)

    rw = aux_ref[:, 0:1]
    cf = aux_ref[:, 1:2] * aux_ref[:, 2:3]
    tz = rw + cf * qsup_ref[...]
    tz = jnp.clip(tz, _V_MIN, _V_MAX)
    b = (tz - _V_MIN) / delta_z
    lo = jnp.floor(b)
    w_lo = p * (lo + 1.0 - b)
    w_hi = p * (b - lo)

    lane_i = jax.lax.broadcasted_iota(jnp.int32, (_ROWS, _AP), 1)
    lane_f = lane_i.astype(f32)

    clo = _cumsum_lane(w_lo, lane_i)
    chi = _cumsum_lane(w_hi, lane_i)

    pos = jnp.where(lo[:, 255:256] <= lane_f,
                    jnp.full((_ROWS, _AP), 256, jnp.int32),
                    jnp.zeros((_ROWS, _AP), jnp.int32))
    for step in (128, 64, 32, 16, 8, 4, 2, 1):
        cand = pos + step
        gl = _take_lane(lo, jnp.bitwise_and(cand - 1, 255))
        ok = jnp.logical_and(cand <= 256, gl <= lane_f)
        pos = jnp.where(ok, cand, pos)

    a_j = pos - 1
    a_jm1 = jnp.where(lane_i == 0, -1, pltpu.roll(a_j, 1, axis=1))

    f_lo = jnp.where(a_j >= 0, _take_lane(clo, jnp.maximum(a_j, 0)), 0.0)
    f_hi = jnp.where(a_jm1 >= 0, _take_lane(chi, jnp.maximum(a_jm1, 0)), 0.0)
    g = f_lo + f_hi
    g_m1 = jnp.where(lane_i == 0, 0.0, pltpu.roll(g, 1, axis=1))
    out_ref[...] = (g - g_m1)[:, :_ATOMS]


@functools.partial(jax.jit, static_argnames=())
def kernel(obs, actions, rewards, bootstrap, discount, q_support,
           W1, b1, g1, be1, W2, b2, g2, be2, W3, b3, g3, be3, W4, b4):
    bsz = obs.shape[0]
    nb = bsz // _ROWS
    f32 = jnp.float32

    aux = jnp.stack([rewards, bootstrap, discount], axis=1)
    qsup = jnp.full((1, _AP), _V_MAX, f32).at[0, :_ATOMS].set(q_support)
    v1 = jnp.stack([b1, g1, be1])
    v2 = jnp.stack([b2, g2, be2])
    v3 = jnp.stack([b3, g3, be3])
    w4p = jnp.zeros((W4.shape[0], _AP), f32).at[:, :_ATOMS].set(W4)
    b4p = jnp.full((1, _AP), -1e30, f32).at[0, :_ATOMS].set(b4)

    const = lambda *shape: pl.BlockSpec(shape, lambda i: (0,) * len(shape))
    grid_spec = pl.GridSpec(
        grid=(nb,),
        in_specs=[
            pl.BlockSpec((_ROWS, 128), lambda i: (i, 0)),
            pl.BlockSpec((_ROWS, 32), lambda i: (i, 0)),
            pl.BlockSpec((_ROWS, 3), lambda i: (i, 0)),
            const(1, _AP),
            const(160, 512), const(3, 512),
            const(512, 256), const(3, 256),
            const(256, 128), const(3, 128),
            const(128, _AP), const(1, _AP),
        ],
        out_specs=pl.BlockSpec((_ROWS, _ATOMS), lambda i: (i, 0)),
    )
    return pl.pallas_call(
        _block_kernel,
        grid_spec=grid_spec,
        out_shape=jax.ShapeDtypeStruct((bsz, _ATOMS), f32),
        compiler_params=pltpu.CompilerParams(
            dimension_semantics=("parallel",),
            vmem_limit_bytes=100 * 1024 * 1024,
        ),
    )(obs, actions, aux, qsup, W1, v1, W2, v2, W3, v3, w4p, b4p)


# speedup vs baseline: 64.7087x; 1.2011x over previous
import functools

import jax
import jax.numpy as jnp
from jax.experimental import pallas as pl
from jax.experimental.pallas import tpu as pltpu

_V_MIN = -10.0
_V_MAX = 10.0
_ATOMS = 251
_AP = 256
_ROWS = 256


def _take_lane(tbl, idx):
    idx_lo = jnp.bitwise_and(idx, 127)
    t0 = jnp.take_along_axis(tbl[:, :128], idx_lo, axis=1)
    t1 = jnp.take_along_axis(tbl[:, 128:], idx_lo, axis=1)
    return jnp.where(idx < 128, t0, t1)


def _cumsum_lane(x, lane_iota):
    for sh in (1, 2, 4, 8, 16, 32, 64, 128):
        rolled = pltpu.roll(x, sh, axis=1)
        x = x + jnp.where(lane_iota >= sh, rolled, 0.0)
    return x


def _ln_silu(h, vecs):
    h = h + vecs[0:1, :]
    m = jnp.mean(h, axis=-1, keepdims=True)
    d = h - m
    v = jnp.mean(d * d, axis=-1, keepdims=True)
    h = d * jax.lax.rsqrt(v + 1e-5) * vecs[1:2, :] + vecs[2:3, :]
    return h * (1.0 / (1.0 + jnp.exp(-h)))


def _block_kernel(obs_ref, act_ref, aux_ref, qsup_ref,
                  w1_ref, v1_ref, w2_ref, v2_ref, w3_ref, v3_ref,
                  w4_ref, b4_ref, out_ref):
    f32 = jnp.float32
    delta_z = (_V_MAX - _V_MIN) / (_ATOMS - 1)

    x = jnp.dot(obs_ref[...], w1_ref[:128, :], preferred_element_type=f32)
    x = x + jnp.dot(act_ref[...], w1_ref[128:, :], preferred_element_type=f32)
    x = _ln_silu(x, v1_ref[...])
    x = _ln_silu(jnp.dot(x, w2_ref[...], preferred_element_type=f32), v2_ref[...])
    x = _ln_silu(jnp.dot(x, w3_ref[...], preferred_element_type=f32), v3_ref[...])
    logits = jnp.dot(x, w4_ref[...], preferred_element_type=f32) + b4_ref[...]

    mx = jnp.max(logits, axis=-1, keepdims=True)
    e = jnp.exp(logits - mx)
    p = e / jnp.sum(e, axis=-1, keepdims=True)

    rw = aux_ref[:, 0:1]
    cf = aux_ref[:, 1:2] * aux_ref[:, 2:3]
    tz = rw + cf * qsup_ref[...]
    tz = jnp.clip(tz, _V_MIN, _V_MAX)
    b = (tz - _V_MIN) / delta_z
    lo = jnp.floor(b)
    w_lo = p * (lo + 1.0 - b)
    w_hi = p * (b - lo)

    lane_i = jax.lax.broadcasted_iota(jnp.int32, (_ROWS, _AP), 1)
    lane_f = lane_i.astype(f32)

    clo = _cumsum_lane(w_lo, lane_i)
    chi = _cumsum_lane(w_hi, lane_i)

    jp1 = lane_f + 1.0
    inv_c = 1.0 / cf
    beta = (rw - 10.0 * cf + 10.0) * 12.5
    t = jnp.clip((jp1 - beta) * inv_c, 0.0, 251.0)
    n = jnp.ceil(t).astype(jnp.int32)
    n = jnp.where(cf == 0.0,
                  jnp.where(lo[:, 0:1] <= lane_f, 251, 0), n)
    n = jnp.where(lane_i >= 250, 251, n)
    for _ in range(2):
        g_up = _take_lane(lo, jnp.minimum(n, 250))
        g_dn = _take_lane(lo, jnp.maximum(n - 1, 0))
        up = jnp.logical_and(n <= 250, g_up <= lane_f)
        dn = jnp.logical_and(n >= 1, g_dn > lane_f)
        n = n + up.astype(jnp.int32) - dn.astype(jnp.int32)
    pos = jnp.where(n > 250, 256, n)

    a_j = pos - 1
    a_jm1 = jnp.where(lane_i == 0, -1, pltpu.roll(a_j, 1, axis=1))

    f_lo = jnp.where(a_j >= 0, _take_lane(clo, jnp.maximum(a_j, 0)), 0.0)
    f_hi = jnp.where(a_jm1 >= 0, _take_lane(chi, jnp.maximum(a_jm1, 0)), 0.0)
    g = f_lo + f_hi
    g_m1 = jnp.where(lane_i == 0, 0.0, pltpu.roll(g, 1, axis=1))
    out_ref[...] = (g - g_m1)[:, :_ATOMS]


@functools.partial(jax.jit, static_argnames=())
def kernel(obs, actions, rewards, bootstrap, discount, q_support,
           W1, b1, g1, be1, W2, b2, g2, be2, W3, b3, g3, be3, W4, b4):
    bsz = obs.shape[0]
    nb = bsz // _ROWS
    f32 = jnp.float32

    aux = jnp.stack([rewards, bootstrap, discount], axis=1)
    qsup = jnp.full((1, _AP), _V_MAX, f32).at[0, :_ATOMS].set(q_support)
    v1 = jnp.stack([b1, g1, be1])
    v2 = jnp.stack([b2, g2, be2])
    v3 = jnp.stack([b3, g3, be3])
    w4p = jnp.zeros((W4.shape[0], _AP), f32).at[:, :_ATOMS].set(W4)
    b4p = jnp.full((1, _AP), -1e30, f32).at[0, :_ATOMS].set(b4)

    const = lambda *shape: pl.BlockSpec(shape, lambda i: (0,) * len(shape))
    grid_spec = pl.GridSpec(
        grid=(nb,),
        in_specs=[
            pl.BlockSpec((_ROWS, 128), lambda i: (i, 0)),
            pl.BlockSpec((_ROWS, 32), lambda i: (i, 0)),
            pl.BlockSpec((_ROWS, 3), lambda i: (i, 0)),
            const(1, _AP),
            const(160, 512), const(3, 512),
            const(512, 256), const(3, 256),
            const(256, 128), const(3, 128),
            const(128, _AP), const(1, _AP),
        ],
        out_specs=pl.BlockSpec((_ROWS, _ATOMS), lambda i: (i, 0)),
    )
    return pl.pallas_call(
        _block_kernel,
        grid_spec=grid_spec,
        out_shape=jax.ShapeDtypeStruct((bsz, _ATOMS), f32),
        compiler_params=pltpu.CompilerParams(
            dimension_semantics=("parallel",),
            vmem_limit_bytes=100 * 1024 * 1024,
        ),
    )(obs, actions, aux, qsup, W1, v1, W2, v2, W3, v3, w4p, b4p)
